# Initial kernel scaffold; baseline (speedup 1.0000x reference)
import jax
import jax.numpy as jnp
from jax.experimental import pallas as pl


def kernel(ty_prob, ty_true):
    raise NotImplementedError("write your pallas kernel here")


# speedup vs baseline: 2.6626x; 2.6626x over previous
import math

import jax
import jax.numpy as jnp
from jax import lax
from jax.experimental import pallas as pl
from jax.experimental.pallas import tpu as pltpu
from jax.experimental.pallas import tpu_sc as plsc

N_CLASSES = 32768
N_ROWS = 4096
SV = 0.1 / (N_CLASSES - 2)
CONF = 0.9
DELTA = CONF - SV
C_ROW = (N_CLASSES - 1) * SV * math.log(SV) + CONF * math.log(CONF)

BR = 256
BC = 4096


def _rowsum_body(x_ref, o_ref):
    j = pl.program_id(1)
    s = jnp.sum(x_ref[...], axis=1)

    @pl.when(j == 0)
    def _():
        o_ref[...] = s

    @pl.when(j > 0)
    def _():
        o_ref[...] = o_ref[...] + s


def _row_sums(ty_prob):
    return pl.pallas_call(
        _rowsum_body,
        grid=(N_ROWS // BR, N_CLASSES // BC),
        in_specs=[pl.BlockSpec((BR, BC), lambda i, j: (i, j))],
        out_specs=pl.BlockSpec((BR,), lambda i, j: (i,)),
        out_shape=jax.ShapeDtypeStruct((N_ROWS,), jnp.float32),
    )(ty_prob)


_NC, _NS, _L = 2, 16, 16
_NW = _NC * _NS
_BPW = N_ROWS // _NW


def _sc_gather(flat_prob, ty_true):
    mesh = plsc.VectorSubcoreMesh(core_axis_name="c", subcore_axis_name="s")

    @pl.kernel(
        out_type=jax.ShapeDtypeStruct((N_ROWS,), jnp.float32),
        mesh=mesh,
        scratch_types=[
            pltpu.VMEM((_BPW,), jnp.int32),
            pltpu.VMEM((_BPW,), jnp.float32),
            pltpu.SemaphoreType.DMA,
        ],
    )
    def k(table_hbm, idx_hbm, out_hbm, idx_v, vals_v, sem):
        wid = lax.axis_index("s") * _NC + lax.axis_index("c")
        base = wid * _BPW
        pltpu.sync_copy(idx_hbm.at[pl.ds(base, _BPW)], idx_v)
        for c in range(_BPW // _L):
            sl = pl.ds(c * _L, _L)
            rows = (base + c * _L) + lax.iota(jnp.int32, _L)
            idx_v[sl] = idx_v[sl] + rows * N_CLASSES
        pltpu.async_copy(table_hbm.at[idx_v], vals_v, sem).wait()
        pltpu.sync_copy(vals_v, out_hbm.at[pl.ds(base, _BPW)])

    return k(flat_prob, ty_true)


def _final_body(tt_ref, rs_ref, pk_ref, o_ref):
    mask = (tt_ref[...] != 0).astype(jnp.float32)
    contrib = mask * (C_ROW - SV * rs_ref[...] - DELTA * pk_ref[...])
    o_ref[...] = jnp.sum(contrib)[None, None]


def _finalize(ty_true, row_sums, picked):
    out = pl.pallas_call(
        _final_body,
        out_shape=jax.ShapeDtypeStruct((1, 1), jnp.float32),
    )(
        ty_true.reshape(1, N_ROWS),
        row_sums.reshape(1, N_ROWS),
        picked.reshape(1, N_ROWS),
    )
    return out[0, 0]


def kernel(ty_prob, ty_true):
    row_sums = _row_sums(ty_prob)
    picked = _sc_gather(ty_prob.reshape(-1), ty_true)
    return _finalize(ty_true, row_sums, picked)


# speedup vs baseline: 7.6939x; 2.8896x over previous
import math

import jax
import jax.numpy as jnp
from jax import lax
from jax.experimental import pallas as pl
from jax.experimental.pallas import tpu as pltpu
from jax.experimental.pallas import tpu_sc as plsc

N_CLASSES = 32768
N_ROWS = 4096
SV = 0.1 / (N_CLASSES - 2)
CONF = 0.9
DELTA = CONF - SV
C_ROW = (N_CLASSES - 1) * SV * math.log(SV) + CONF * math.log(CONF)

BR = 256
BC = 4096


def _rowsum_body(x_ref, o_ref):
    j = pl.program_id(1)
    s = jnp.sum(x_ref[...], axis=1)

    @pl.when(j == 0)
    def _():
        o_ref[...] = s

    @pl.when(j > 0)
    def _():
        o_ref[...] = o_ref[...] + s


def _row_sums(ty_prob):
    return pl.pallas_call(
        _rowsum_body,
        grid=(N_ROWS // BR, N_CLASSES // BC),
        in_specs=[pl.BlockSpec((BR, BC), lambda i, j: (i, j))],
        out_specs=pl.BlockSpec((BR,), lambda i, j: (i,)),
        out_shape=jax.ShapeDtypeStruct((N_ROWS,), jnp.float32),
    )(ty_prob)


_NC, _NS, _L = 2, 16, 16
_NW = _NC * _NS
_BPW = N_ROWS // _NW


def _sc_gather(flat_prob, ty_true):
    mesh = plsc.VectorSubcoreMesh(core_axis_name="c", subcore_axis_name="s")

    @pl.kernel(
        out_type=jax.ShapeDtypeStruct((N_ROWS,), jnp.float32),
        mesh=mesh,
        scratch_types=[
            pltpu.VMEM((_BPW,), jnp.int32),
            pltpu.VMEM((_BPW,), jnp.float32),
            pltpu.SemaphoreType.DMA,
        ],
    )
    def k(table_hbm, idx_hbm, out_hbm, idx_v, vals_v, sem):
        wid = lax.axis_index("s") * _NC + lax.axis_index("c")
        base = wid * _BPW
        pltpu.sync_copy(idx_hbm.at[pl.ds(base, _BPW)], idx_v)
        for c in range(_BPW // _L):
            sl = pl.ds(c * _L, _L)
            i_vec = (base + c * _L) + lax.iota(jnp.int32, _L)
            j_vec = idx_v[sl]
            idx_v[sl] = (
                (i_vec >> 3) * ((N_CLASSES // 128) * 1024)
                + (j_vec >> 7) * 1024
                + (i_vec & 7) * 128
                + (j_vec & 127)
            )
        pltpu.async_copy(table_hbm.at[idx_v], vals_v, sem).wait()
        pltpu.sync_copy(vals_v, out_hbm.at[pl.ds(base, _BPW)])

    return k(flat_prob, ty_true)


def _final_body(tt_ref, rs_ref, pk_ref, o_ref):
    mask = (tt_ref[...] != 0).astype(jnp.float32)
    contrib = mask * (C_ROW - SV * rs_ref[...] - DELTA * pk_ref[...])
    o_ref[...] = jnp.sum(contrib)[None, None]


def _finalize(ty_true, row_sums, picked):
    out = pl.pallas_call(
        _final_body,
        out_shape=jax.ShapeDtypeStruct((1, 1), jnp.float32),
    )(
        ty_true.reshape(1, N_ROWS),
        row_sums.reshape(1, N_ROWS),
        picked.reshape(1, N_ROWS),
    )
    return out[0, 0]


def kernel(ty_prob, ty_true):
    row_sums = _row_sums(ty_prob)
    tiled_flat = (
        ty_prob.reshape(N_ROWS // 8, 8, N_CLASSES // 128, 128)
        .transpose(0, 2, 1, 3)
        .reshape(-1)
    )
    picked = _sc_gather(tiled_flat, ty_true)
    return _finalize(ty_true, row_sums, picked)


# speedup vs baseline: 7.7090x; 1.0020x over previous
import math

import jax
import jax.numpy as jnp
from jax import lax
from jax.experimental import pallas as pl
from jax.experimental.pallas import tpu as pltpu
from jax.experimental.pallas import tpu_sc as plsc

N_CLASSES = 32768
N_ROWS = 4096
SV = 0.1 / (N_CLASSES - 2)
CONF = 0.9
DELTA = CONF - SV
C_ROW = (N_CLASSES - 1) * SV * math.log(SV) + CONF * math.log(CONF)

BR = 256
BC = 4096


def _rowsum_body(x_ref, o_ref):
    j = pl.program_id(1)
    acc = x_ref[:, 0:128]
    for c in range(1, BC // 128):
        acc = acc + x_ref[:, c * 128:(c + 1) * 128]

    @pl.when(j == 0)
    def _():
        o_ref[...] = acc

    @pl.when(j > 0)
    def _():
        o_ref[...] = o_ref[...] + acc


def _row_sums(ty_prob):
    return pl.pallas_call(
        _rowsum_body,
        grid=(N_ROWS // BR, N_CLASSES // BC),
        in_specs=[pl.BlockSpec((BR, BC), lambda i, j: (i, j))],
        out_specs=pl.BlockSpec((BR, 128), lambda i, j: (i, 0)),
        out_shape=jax.ShapeDtypeStruct((N_ROWS, 128), jnp.float32),
    )(ty_prob)


_NC, _NS, _L = 2, 16, 16
_NW = _NC * _NS
_BPW = N_ROWS // _NW


def _sc_gather(flat_prob, ty_true):
    mesh = plsc.VectorSubcoreMesh(core_axis_name="c", subcore_axis_name="s")

    @pl.kernel(
        out_type=jax.ShapeDtypeStruct((N_ROWS,), jnp.float32),
        mesh=mesh,
        scratch_types=[
            pltpu.VMEM((_BPW,), jnp.int32),
            pltpu.VMEM((_BPW,), jnp.float32),
            pltpu.SemaphoreType.DMA,
        ],
    )
    def k(table_hbm, idx_hbm, out_hbm, idx_v, vals_v, sem):
        wid = lax.axis_index("s") * _NC + lax.axis_index("c")
        base = wid * _BPW
        pltpu.sync_copy(idx_hbm.at[pl.ds(base, _BPW)], idx_v)
        for c in range(_BPW // _L):
            sl = pl.ds(c * _L, _L)
            i_vec = (base + c * _L) + lax.iota(jnp.int32, _L)
            j_vec = idx_v[sl]
            idx_v[sl] = (
                (i_vec >> 3) * ((N_CLASSES // 128) * 1024)
                + (j_vec >> 7) * 1024
                + (i_vec & 7) * 128
                + (j_vec & 127)
            )
        pltpu.async_copy(table_hbm.at[idx_v], vals_v, sem).wait()
        pltpu.sync_copy(vals_v, out_hbm.at[pl.ds(base, _BPW)])

    return k(flat_prob, ty_true)


def _final_body(tt_ref, rs_ref, pk_ref, o_ref):
    mask = (tt_ref[...] != 0).astype(jnp.float32)
    n_live = jnp.sum(mask)
    masked_rowsum = jnp.sum(mask * rs_ref[...])
    masked_picked = jnp.sum(mask * pk_ref[...])
    loss = C_ROW * n_live - SV * masked_rowsum - DELTA * masked_picked
    o_ref[...] = loss[None, None]


def _finalize(ty_true, row_sums128, picked):
    out = pl.pallas_call(
        _final_body,
        out_shape=jax.ShapeDtypeStruct((1, 1), jnp.float32),
    )(
        ty_true.reshape(N_ROWS, 1),
        row_sums128,
        picked.reshape(N_ROWS, 1),
    )
    return out[0, 0]


def kernel(ty_prob, ty_true):
    row_sums = _row_sums(ty_prob)
    tiled_flat = (
        ty_prob.reshape(N_ROWS // 8, 8, N_CLASSES // 128, 128)
        .transpose(0, 2, 1, 3)
        .reshape(-1)
    )
    picked = _sc_gather(tiled_flat, ty_true)
    return _finalize(ty_true, row_sums, picked)


# speedup vs baseline: 7.8712x; 1.0210x over previous
import math

import jax
import jax.numpy as jnp
from jax import lax
from jax.experimental import pallas as pl
from jax.experimental.pallas import tpu as pltpu
from jax.experimental.pallas import tpu_sc as plsc

N_CLASSES = 32768
N_ROWS = 4096
SV = 0.1 / (N_CLASSES - 2)
CONF = 0.9
DELTA = CONF - SV
C_ROW = (N_CLASSES - 1) * SV * math.log(SV) + CONF * math.log(CONF)

BR = 512
BC = 8192


def _rowsum_body(x_ref, o_ref):
    j = pl.program_id(1)
    acc = x_ref[:, 0:128]
    for c in range(1, BC // 128):
        acc = acc + x_ref[:, c * 128:(c + 1) * 128]

    @pl.when(j == 0)
    def _():
        o_ref[...] = acc

    @pl.when(j > 0)
    def _():
        o_ref[...] = o_ref[...] + acc


def _row_sums(ty_prob):
    return pl.pallas_call(
        _rowsum_body,
        grid=(N_ROWS // BR, N_CLASSES // BC),
        in_specs=[pl.BlockSpec((BR, BC), lambda i, j: (i, j))],
        out_specs=pl.BlockSpec((BR, 128), lambda i, j: (i, 0)),
        out_shape=jax.ShapeDtypeStruct((N_ROWS, 128), jnp.float32),
    )(ty_prob)


_NC, _NS, _L = 2, 16, 16
_NW = _NC * _NS
_BPW = N_ROWS // _NW


def _sc_gather(flat_prob, ty_true):
    mesh = plsc.VectorSubcoreMesh(core_axis_name="c", subcore_axis_name="s")

    @pl.kernel(
        out_type=jax.ShapeDtypeStruct((N_ROWS,), jnp.float32),
        mesh=mesh,
        scratch_types=[
            pltpu.VMEM((_BPW,), jnp.int32),
            pltpu.VMEM((_BPW,), jnp.float32),
            pltpu.SemaphoreType.DMA,
        ],
    )
    def k(table_hbm, idx_hbm, out_hbm, idx_v, vals_v, sem):
        wid = lax.axis_index("s") * _NC + lax.axis_index("c")
        base = wid * _BPW
        pltpu.sync_copy(idx_hbm.at[pl.ds(base, _BPW)], idx_v)
        for c in range(_BPW // _L):
            sl = pl.ds(c * _L, _L)
            i_vec = (base + c * _L) + lax.iota(jnp.int32, _L)
            j_vec = idx_v[sl]
            idx_v[sl] = (
                (i_vec >> 3) * ((N_CLASSES // 128) * 1024)
                + (j_vec >> 7) * 1024
                + (i_vec & 7) * 128
                + (j_vec & 127)
            )
        pltpu.async_copy(table_hbm.at[idx_v], vals_v, sem).wait()
        pltpu.sync_copy(vals_v, out_hbm.at[pl.ds(base, _BPW)])

    return k(flat_prob, ty_true)


def _final_body(tt_ref, rs_ref, pk_ref, o_ref):
    mask = (tt_ref[...] != 0).astype(jnp.float32)
    n_live = jnp.sum(mask)
    masked_rowsum = jnp.sum(mask * rs_ref[...])
    masked_picked = jnp.sum(mask * pk_ref[...])
    loss = C_ROW * n_live - SV * masked_rowsum - DELTA * masked_picked
    o_ref[...] = loss[None, None]


def _finalize(ty_true, row_sums128, picked):
    out = pl.pallas_call(
        _final_body,
        out_shape=jax.ShapeDtypeStruct((1, 1), jnp.float32),
    )(
        ty_true.reshape(N_ROWS, 1),
        row_sums128,
        picked.reshape(N_ROWS, 1),
    )
    return out[0, 0]


def kernel(ty_prob, ty_true):
    row_sums = _row_sums(ty_prob)
    tiled_flat = (
        ty_prob.reshape(N_ROWS // 8, 8, N_CLASSES // 128, 128)
        .transpose(0, 2, 1, 3)
        .reshape(-1)
    )
    picked = _sc_gather(tiled_flat, ty_true)
    return _finalize(ty_true, row_sums, picked)


# speedup vs baseline: 7.8726x; 1.0002x over previous
import math

import jax
import jax.numpy as jnp
from jax import lax
from jax.experimental import pallas as pl
from jax.experimental.pallas import tpu as pltpu
from jax.experimental.pallas import tpu_sc as plsc

N_CLASSES = 32768
N_ROWS = 4096
SV = 0.1 / (N_CLASSES - 2)
CONF = 0.9
DELTA = CONF - SV
C_ROW = (N_CLASSES - 1) * SV * math.log(SV) + CONF * math.log(CONF)

BR = 512
BC = 8192


def _rowsum_body(x_ref, o_ref):
    j = pl.program_id(1)
    acc = x_ref[:, 0:128]
    for c in range(1, BC // 128):
        acc = acc + x_ref[:, c * 128:(c + 1) * 128]

    @pl.when(j == 0)
    def _():
        o_ref[...] = acc

    @pl.when(j > 0)
    def _():
        o_ref[...] = o_ref[...] + acc


def _row_sums(ty_prob):
    return pl.pallas_call(
        _rowsum_body,
        grid=(N_ROWS // BR, N_CLASSES // BC),
        in_specs=[pl.BlockSpec((BR, BC), lambda i, j: (i, j))],
        out_specs=pl.BlockSpec((BR, 128), lambda i, j: (i, 0)),
        out_shape=jax.ShapeDtypeStruct((N_ROWS, 128), jnp.float32),
        compiler_params=pltpu.CompilerParams(
            dimension_semantics=("parallel", "arbitrary"),
        ),
    )(ty_prob)


_NC, _NS, _L = 2, 16, 16
_NW = _NC * _NS
_BPW = N_ROWS // _NW


def _sc_gather(flat_prob, ty_true):
    mesh = plsc.VectorSubcoreMesh(core_axis_name="c", subcore_axis_name="s")

    @pl.kernel(
        out_type=jax.ShapeDtypeStruct((N_ROWS,), jnp.float32),
        mesh=mesh,
        scratch_types=[
            pltpu.VMEM((_BPW,), jnp.int32),
            pltpu.VMEM((_BPW,), jnp.float32),
            pltpu.SemaphoreType.DMA,
        ],
    )
    def k(table_hbm, idx_hbm, out_hbm, idx_v, vals_v, sem):
        wid = lax.axis_index("s") * _NC + lax.axis_index("c")
        base = wid * _BPW
        pltpu.sync_copy(idx_hbm.at[pl.ds(base, _BPW)], idx_v)
        for c in range(_BPW // _L):
            sl = pl.ds(c * _L, _L)
            i_vec = (base + c * _L) + lax.iota(jnp.int32, _L)
            j_vec = idx_v[sl]
            idx_v[sl] = (
                (i_vec >> 3) * ((N_CLASSES // 128) * 1024)
                + (j_vec >> 7) * 1024
                + (i_vec & 7) * 128
                + (j_vec & 127)
            )
        pltpu.async_copy(table_hbm.at[idx_v], vals_v, sem).wait()
        pltpu.sync_copy(vals_v, out_hbm.at[pl.ds(base, _BPW)])

    return k(flat_prob, ty_true)


def _final_body(tt_ref, rs_ref, pk_ref, o_ref):
    mask = (tt_ref[...] != 0).astype(jnp.float32)
    n_live = jnp.sum(mask)
    masked_rowsum = jnp.sum(mask * rs_ref[...])
    masked_picked = jnp.sum(mask * pk_ref[...])
    loss = C_ROW * n_live - SV * masked_rowsum - DELTA * masked_picked
    o_ref[...] = loss[None, None]


def _finalize(ty_true, row_sums128, picked):
    out = pl.pallas_call(
        _final_body,
        out_shape=jax.ShapeDtypeStruct((1, 1), jnp.float32),
    )(
        ty_true.reshape(N_ROWS, 1),
        row_sums128,
        picked.reshape(N_ROWS, 1),
    )
    return out[0, 0]


def kernel(ty_prob, ty_true):
    row_sums = _row_sums(ty_prob)
    tiled_flat = (
        ty_prob.reshape(N_ROWS // 8, 8, N_CLASSES // 128, 128)
        .transpose(0, 2, 1, 3)
        .reshape(-1)
    )
    picked = _sc_gather(tiled_flat, ty_true)
    return _finalize(ty_true, row_sums, picked)


# speedup vs baseline: 8.3410x; 1.0595x over previous
import math

import jax
import jax.numpy as jnp
from jax import lax
from jax.experimental import pallas as pl
from jax.experimental.pallas import tpu as pltpu
from jax.experimental.pallas import tpu_sc as plsc

N_CLASSES = 32768
N_ROWS = 4096
SV = 0.1 / (N_CLASSES - 2)
CONF = 0.9
DELTA = CONF - SV
C_ROW = (N_CLASSES - 1) * SV * math.log(SV) + CONF * math.log(CONF)

BR = 128


def _fused_body(tt_ref, pk_ref, x_ref, o_ref):
    i = pl.program_id(0)
    acc = x_ref[:, 0:128]
    for c in range(1, N_CLASSES // 128):
        acc = acc + x_ref[:, c * 128:(c + 1) * 128]
    rows = jnp.sum(acc, axis=1, keepdims=True)
    mask = (tt_ref[...] != 0).astype(jnp.float32)
    blk = jnp.sum(mask * (C_ROW - SV * rows - DELTA * pk_ref[...]))

    @pl.when(i == 0)
    def _():
        o_ref[...] = blk[None, None]

    @pl.when(i > 0)
    def _():
        o_ref[...] = o_ref[...] + blk[None, None]


def _fused_loss(ty_true, picked, ty_prob):
    out = pl.pallas_call(
        _fused_body,
        grid=(N_ROWS // BR,),
        in_specs=[
            pl.BlockSpec((BR, 1), lambda i: (i, 0)),
            pl.BlockSpec((BR, 1), lambda i: (i, 0)),
            pl.BlockSpec((BR, N_CLASSES), lambda i: (i, 0)),
        ],
        out_specs=pl.BlockSpec((1, 1), lambda i: (0, 0)),
        out_shape=jax.ShapeDtypeStruct((1, 1), jnp.float32),
    )(ty_true.reshape(N_ROWS, 1), picked.reshape(N_ROWS, 1), ty_prob)
    return out[0, 0]


_NC, _NS, _L = 2, 16, 16
_NW = _NC * _NS
_BPW = N_ROWS // _NW


def _sc_gather(flat_prob, ty_true):
    mesh = plsc.VectorSubcoreMesh(core_axis_name="c", subcore_axis_name="s")

    @pl.kernel(
        out_type=jax.ShapeDtypeStruct((N_ROWS,), jnp.float32),
        mesh=mesh,
        scratch_types=[
            pltpu.VMEM((_BPW,), jnp.int32),
            pltpu.VMEM((_BPW,), jnp.float32),
            pltpu.SemaphoreType.DMA,
        ],
    )
    def k(table_hbm, idx_hbm, out_hbm, idx_v, vals_v, sem):
        wid = lax.axis_index("s") * _NC + lax.axis_index("c")
        base = wid * _BPW
        pltpu.sync_copy(idx_hbm.at[pl.ds(base, _BPW)], idx_v)
        for c in range(_BPW // _L):
            sl = pl.ds(c * _L, _L)
            i_vec = (base + c * _L) + lax.iota(jnp.int32, _L)
            j_vec = idx_v[sl]
            idx_v[sl] = (
                (i_vec >> 3) * ((N_CLASSES // 128) * 1024)
                + (j_vec >> 7) * 1024
                + (i_vec & 7) * 128
                + (j_vec & 127)
            )
        pltpu.async_copy(table_hbm.at[idx_v], vals_v, sem).wait()
        pltpu.sync_copy(vals_v, out_hbm.at[pl.ds(base, _BPW)])

    return k(flat_prob, ty_true)


def kernel(ty_prob, ty_true):
    tiled_flat = (
        ty_prob.reshape(N_ROWS // 8, 8, N_CLASSES // 128, 128)
        .transpose(0, 2, 1, 3)
        .reshape(-1)
    )
    picked = _sc_gather(tiled_flat, ty_true)
    return _fused_loss(ty_true, picked, ty_prob)


# speedup vs baseline: 8.3717x; 1.0037x over previous
import math

import jax
import jax.numpy as jnp
from jax import lax
from jax.experimental import pallas as pl
from jax.experimental.pallas import tpu as pltpu
from jax.experimental.pallas import tpu_sc as plsc

N_CLASSES = 32768
N_ROWS = 4096
SV = 0.1 / (N_CLASSES - 2)
CONF = 0.9
DELTA = CONF - SV
C_ROW = (N_CLASSES - 1) * SV * math.log(SV) + CONF * math.log(CONF)

BR = 128


def _fused_body(tt_ref, pk_ref, x_ref, o_ref):
    i = pl.program_id(0)
    acc = x_ref[:, 0:128]
    for c in range(1, N_CLASSES // 128):
        acc = acc + x_ref[:, c * 128:(c + 1) * 128]
    rows = jnp.sum(acc, axis=1, keepdims=True)
    tt = tt_ref[pl.ds(i * BR, BR), :]
    pk = pk_ref[pl.ds(i * BR, BR), :]
    mask = (tt != 0).astype(jnp.float32)
    blk = jnp.sum(mask * (C_ROW - SV * rows - DELTA * pk))

    @pl.when(i == 0)
    def _():
        o_ref[...] = blk[None, None]

    @pl.when(i > 0)
    def _():
        o_ref[...] = o_ref[...] + blk[None, None]


def _fused_loss(ty_true, picked, ty_prob):
    out = pl.pallas_call(
        _fused_body,
        grid=(N_ROWS // BR,),
        in_specs=[
            pl.BlockSpec((N_ROWS, 1), lambda i: (0, 0)),
            pl.BlockSpec((N_ROWS, 1), lambda i: (0, 0)),
            pl.BlockSpec((BR, N_CLASSES), lambda i: (i, 0)),
        ],
        out_specs=pl.BlockSpec((1, 1), lambda i: (0, 0)),
        out_shape=jax.ShapeDtypeStruct((1, 1), jnp.float32),
    )(ty_true.reshape(N_ROWS, 1), picked.reshape(N_ROWS, 1), ty_prob)
    return out[0, 0]


_NC, _NS, _L = 2, 16, 16
_NW = _NC * _NS
_BPW = N_ROWS // _NW


def _sc_gather(flat_prob, ty_true):
    mesh = plsc.VectorSubcoreMesh(core_axis_name="c", subcore_axis_name="s")

    @pl.kernel(
        out_type=jax.ShapeDtypeStruct((N_ROWS,), jnp.float32),
        mesh=mesh,
        scratch_types=[
            pltpu.VMEM((_BPW,), jnp.int32),
            pltpu.VMEM((_BPW,), jnp.float32),
            pltpu.SemaphoreType.DMA,
        ],
    )
    def k(table_hbm, idx_hbm, out_hbm, idx_v, vals_v, sem):
        wid = lax.axis_index("s") * _NC + lax.axis_index("c")
        base = wid * _BPW
        pltpu.sync_copy(idx_hbm.at[pl.ds(base, _BPW)], idx_v)
        for c in range(_BPW // _L):
            sl = pl.ds(c * _L, _L)
            i_vec = (base + c * _L) + lax.iota(jnp.int32, _L)
            j_vec = idx_v[sl]
            idx_v[sl] = (
                (i_vec >> 3) * ((N_CLASSES // 128) * 1024)
                + (j_vec >> 7) * 1024
                + (i_vec & 7) * 128
                + (j_vec & 127)
            )
        pltpu.async_copy(table_hbm.at[idx_v], vals_v, sem).wait()
        pltpu.sync_copy(vals_v, out_hbm.at[pl.ds(base, _BPW)])

    return k(flat_prob, ty_true)


def kernel(ty_prob, ty_true):
    tiled_flat = (
        ty_prob.reshape(N_ROWS // 8, 8, N_CLASSES // 128, 128)
        .transpose(0, 2, 1, 3)
        .reshape(-1)
    )
    picked = _sc_gather(tiled_flat, ty_true)
    return _fused_loss(ty_true, picked, ty_prob)
